# baseline (speedup 1.0000x reference)
import jax
import jax.numpy as jnp
from jax import lax
from jax.experimental import pallas as pl
from jax.experimental.pallas import tpu as pltpu
from jax.experimental.pallas import tpu_sc as plsc

_EPS = 1e-3
_H = 768
_NLANE = 16
_NJ = _H // _NLANE
_NW = 32
_B = 4
_S = 2048
_PPW = _S // _NW
_CP = 8
_NCHUNK = _PPW // _CP


def _shuf(v, idx):
    dnums = lax.GatherDimensionNumbers(
        offset_dims=(), collapsed_slice_dims=(0,), start_index_map=(0,))
    return lax.gather(v, idx[:, None], dnums, (1,),
                      mode=lax.GatherScatterMode.PROMISE_IN_BOUNDS)


def _lane_sum(v):
    lanes = lax.iota(jnp.int32, _NLANE)
    for k in (1, 2, 4, 8):
        v = v + _shuf(v, lanes ^ k)
    return v


def _nr_rsqrt(v):
    i = lax.bitcast_convert_type(v, jnp.int32)
    i = jnp.int32(0x5F3759DF) - lax.shift_right_arithmetic(i, 1)
    y = lax.bitcast_convert_type(i, jnp.float32)
    for _ in range(3):
        y = y * (1.5 - 0.5 * v * y * y)
    return y


def _sc_body(x_hbm, pos_hbm, g_hbm, b_hbm, out_hbm,
             xb0, xb1, pb0, pb1, ob0, ob1, g_v, b_v,
             si0, si1, so0, so1):
    wid = lax.axis_index("s") * 2 + lax.axis_index("c")
    pbase = wid * _PPW
    pltpu.sync_copy(g_hbm, g_v)
    pltpu.sync_copy(b_hbm, b_v)

    def in_copies(c, xb, pb):
        pstart = pbase + c * _CP
        yield pos_hbm.at[pl.ds(pstart, _CP)], pb
        for b in range(_B):
            yield x_hbm.at[pl.ds(b * _S + pstart, _CP)], xb.at[b]

    def issue_in(c, xb, pb, sem):
        for src, dst in in_copies(c, xb, pb):
            pltpu.async_copy(src, dst, sem)

    def wait_in(c, xb, pb, sem):
        for src, dst in in_copies(c, xb, pb):
            pltpu.make_async_copy(src, dst, sem).wait()

    def out_copies(c, ob):
        pstart = pbase + c * _CP
        for b in range(_B):
            yield ob.at[b], out_hbm.at[pl.ds(b * _S + pstart, _CP)]

    def issue_out(c, ob, sem):
        for src, dst in out_copies(c, ob):
            pltpu.async_copy(src, dst, sem)

    def wait_out(c, ob, sem):
        for src, dst in out_copies(c, ob):
            pltpu.make_async_copy(src, dst, sem).wait()

    def compute_chunk(xb, pb, ob):
        def pos_body(p, carry):
            s = [jnp.zeros((_NLANE,), jnp.float32) for _ in range(_B)]
            ss = [jnp.zeros((_NLANE,), jnp.float32) for _ in range(_B)]
            for j in range(_NJ):
                sl = pl.ds(j * _NLANE, _NLANE)
                pj = pb[p, sl]
                for b in range(_B):
                    v = xb[b, p, sl] + pj
                    xb[b, p, sl] = v
                    s[b] = s[b] + v
                    ss[b] = ss[b] + v * v
            a1 = []
            a0 = []
            for b in range(_B):
                mean = _lane_sum(s[b]) * (1.0 / _H)
                var = _lane_sum(ss[b]) * (1.0 / _H) - mean * mean
                rinv = _nr_rsqrt(var + _EPS)
                a1.append(rinv)
                a0.append(-mean * rinv)
            for j in range(_NJ):
                sl = pl.ds(j * _NLANE, _NLANE)
                gj = g_v[sl]
                bj = b_v[sl]
                for b in range(_B):
                    h = xb[b, p, sl]
                    ob[b, p, sl] = (h * a1[b] + a0[b]) * gj + bj
            return carry

        lax.fori_loop(0, _CP, pos_body, 0)

    issue_in(0, xb0, pb0, si0)

    def step(sidx, carry):
        c0 = 2 * sidx
        c1 = c0 + 1

        @pl.when(sidx > 0)
        def _():
            wait_out(c0 - 2, ob0, so0)
        issue_in(c1, xb1, pb1, si1)
        wait_in(c0, xb0, pb0, si0)
        compute_chunk(xb0, pb0, ob0)
        issue_out(c0, ob0, so0)

        @pl.when(sidx > 0)
        def _():
            wait_out(c1 - 2, ob1, so1)

        @pl.when(sidx < _NCHUNK // 2 - 1)
        def _():
            issue_in(c0 + 2, xb0, pb0, si0)
        wait_in(c1, xb1, pb1, si1)
        compute_chunk(xb1, pb1, ob1)
        issue_out(c1, ob1, so1)
        return carry

    lax.fori_loop(0, _NCHUNK // 2, step, 0)
    wait_out(_NCHUNK - 2, ob0, so0)
    wait_out(_NCHUNK - 1, ob1, so1)


def kernel(x, pos_table, gamma, beta):
    B, S, H = x.shape
    x2 = x.reshape(B * S, H)
    k = pl.kernel(
        _sc_body,
        out_type=jax.ShapeDtypeStruct((B * S, H), jnp.float32),
        mesh=plsc.VectorSubcoreMesh(core_axis_name="c", subcore_axis_name="s"),
        scratch_types=[
            pltpu.VMEM((_B, _CP, H), jnp.float32),
            pltpu.VMEM((_B, _CP, H), jnp.float32),
            pltpu.VMEM((_CP, H), jnp.float32),
            pltpu.VMEM((_CP, H), jnp.float32),
            pltpu.VMEM((_B, _CP, H), jnp.float32),
            pltpu.VMEM((_B, _CP, H), jnp.float32),
            pltpu.VMEM((H,), jnp.float32),
            pltpu.VMEM((H,), jnp.float32),
            pltpu.SemaphoreType.DMA,
            pltpu.SemaphoreType.DMA,
            pltpu.SemaphoreType.DMA,
            pltpu.SemaphoreType.DMA,
        ],
    )
    out = k(x2, pos_table, gamma, beta)
    return out.reshape(B, S, H)


# speedup vs baseline: 6.8241x; 6.8241x over previous
import jax
import jax.numpy as jnp
from jax.experimental import pallas as pl

_EPS = 1e-3


def _body(x_ref, pos_ref, g_ref, b_ref, o_ref):
    h = x_ref[...] + pos_ref[...][None]
    mean = jnp.mean(h, axis=-1, keepdims=True)
    d = h - mean
    var = jnp.mean(d * d, axis=-1, keepdims=True)
    o_ref[...] = d * jax.lax.rsqrt(var + _EPS) * g_ref[...] + b_ref[...]


def kernel(x, pos_table, gamma, beta):
    B, S, H = x.shape
    blk = 512
    out = pl.pallas_call(
        _body,
        grid=(S // blk,),
        in_specs=[
            pl.BlockSpec((B, blk, H), lambda j: (0, j, 0)),
            pl.BlockSpec((blk, H), lambda j: (j, 0)),
            pl.BlockSpec((1, H), lambda j: (0, 0)),
            pl.BlockSpec((1, H), lambda j: (0, 0)),
        ],
        out_specs=pl.BlockSpec((B, blk, H), lambda j: (0, j, 0)),
        out_shape=jax.ShapeDtypeStruct(x.shape, x.dtype),
    )(x, pos_table, gamma.reshape(1, H), beta.reshape(1, H))
    return out
